# baseline (speedup 1.0000x reference)
import functools

import jax
import jax.numpy as jnp
from jax import lax
from jax.experimental import pallas as pl
from jax.experimental.pallas import tpu as pltpu
from jax.experimental.pallas import tpu_sc as plsc

_NC = 2
_NS = 16
_CHUNK = 80


def _make_sc_scatter_add(n_pad, n_edges, dim):
    epw = n_edges // (_NC * _NS)
    n_chunks = epw // _CHUNK
    rows_per_sub = n_pad // _NS
    zrows = _CHUNK
    n_zero_iters = rows_per_sub // zrows
    mesh = plsc.VectorSubcoreMesh(core_axis_name="c", subcore_axis_name="s")

    @functools.partial(
        pl.kernel,
        out_type=jax.ShapeDtypeStruct((_NC, n_pad, dim), jnp.float32),
        mesh=mesh,
        scratch_types=[
            pltpu.VMEM((epw,), jnp.int32),
            pltpu.VMEM((_CHUNK,), jnp.int32),
            pltpu.VMEM((_CHUNK,), jnp.int32),
            pltpu.VMEM((_CHUNK,), jnp.int32),
            pltpu.VMEM((_CHUNK, dim), jnp.float32),
            pltpu.VMEM((_CHUNK, dim), jnp.float32),
            pltpu.VMEM((_CHUNK, dim), jnp.float32),
            pltpu.VMEM_SHARED((n_pad, dim), jnp.float32),
            pltpu.SemaphoreType.DMA,
            pltpu.SemaphoreType.DMA,
            pltpu.SemaphoreType.DMA,
            pltpu.SemaphoreType.DMA,
            pltpu.SemaphoreType.DMA,
            pltpu.SemaphoreType.DMA,
            pltpu.SemaphoreType.DMA,
            pltpu.SemaphoreType.DMA,
            pltpu.SemaphoreType.DMA,
            pltpu.SemaphoreType.DMA,
        ],
    )
    def k(h_hbm, src_hbm, dst_hbm, out_hbm, src_v, dst_a, dst_b, dst_c,
          rows_a, rows_b, rows_c, acc_sh,
          sem_i, sem_ga, sem_gb, sem_gc, sem_da, sem_db, sem_dc,
          sem_sa, sem_sb, sem_sc):
        c = lax.axis_index("c")
        s = lax.axis_index("s")
        wid = c * _NS + s
        base0 = wid * epw

        dst_bufs = (dst_a, dst_b, dst_c)
        rows_bufs = (rows_a, rows_b, rows_c)
        gsems = (sem_ga, sem_gb, sem_gc)
        dsems = (sem_da, sem_db, sem_dc)
        ssems = (sem_sa, sem_sb, sem_sc)

        def src_idx(t):
            return src_v.at[pl.ds(pl.multiple_of(t * _CHUNK, 8), _CHUNK)]

        def dst_desc(t, j):
            return pltpu.make_async_copy(
                dst_hbm.at[pl.ds(pl.multiple_of(base0 + t * _CHUNK, 8), _CHUNK)],
                dst_bufs[j], dsems[j])

        def gather_desc(t, j):
            return pltpu.make_async_copy(
                h_hbm.at[pl.ds(pl.multiple_of(t * _CHUNK, 8), _CHUNK)],
                rows_bufs[j], gsems[j])

        cp_src = pltpu.async_copy(src_hbm.at[pl.ds(wid * epw, epw)], src_v, sem_i)
        dst_desc(0, 0).start()
        dst_desc(1, 1).start()
        dst_desc(2, 2).start()

        zvec = jnp.zeros((1, 16), jnp.float32)

        @pl.loop(0, _CHUNK)
        def _(r):
            @pl.loop(0, dim, step=16)
            def _(j):
                rows_a.at[pl.ds(r, 1), pl.ds(j, 16)][...] = zvec

        @pl.loop(0, n_zero_iters)
        def _(z):
            pltpu.sync_copy(rows_a, acc_sh.at[pl.ds(s * rows_per_sub + z * zrows, zrows)])

        cp_src.wait()
        plsc.subcore_barrier()

        class _Scat:
            def __init__(self, j):
                self.j = j

            def start(self):
                pass

            def wait(self):
                pass

        def scat_desc(j):
            return _Scat(j)

        gather_desc(0, 0).start()
        gather_desc(1, 1).start()
        gather_desc(2, 2).start()

        def step(t, j, reuse):
            if reuse:
                jp = (j + 2) % 3
                scat_desc(jp).wait()
                dst_desc(t + 2, jp).start()
                gather_desc(t + 2, jp).start()
            gather_desc(t, j).wait()
            dst_desc(t, j).wait()
            scat_desc(j).start()

        step(0, 0, False)

        @pl.loop(0, (n_chunks - 5) // 3)
        def _(kk):
            t = 3 * kk
            step(t + 1, 1, True)
            step(t + 2, 2, True)
            step(t + 3, 0, True)

        step(n_chunks - 4, (n_chunks - 4) % 3, True)
        step(n_chunks - 3, (n_chunks - 3) % 3, True)
        step(n_chunks - 2, (n_chunks - 2) % 3, False)
        step(n_chunks - 1, (n_chunks - 1) % 3, False)

        scat_desc((n_chunks - 3) % 3).wait()
        scat_desc((n_chunks - 2) % 3).wait()
        scat_desc((n_chunks - 1) % 3).wait()

        plsc.subcore_barrier()

        pltpu.sync_copy(
            acc_sh.at[pl.ds(s * rows_per_sub, rows_per_sub)],
            out_hbm.at[c, pl.ds(s * rows_per_sub, rows_per_sub)],
        )

    return k


def _mlp1_body(x_ref, agg_ref, W1_ref, b1_ref, W2_ref, b2_ref, o_ref):
    z = x_ref[...] + agg_ref[0] + agg_ref[1]
    t = jnp.dot(z, W1_ref[...], preferred_element_type=jnp.float32) + b1_ref[...]
    t = jnp.maximum(t, 0.0)
    h = jnp.dot(t, W2_ref[...], preferred_element_type=jnp.float32) + b2_ref[...]
    o_ref[...] = jnp.maximum(h, 0.0)


def _mlp2_pool_body(batch_ref, h_ref, agg_ref, W1_ref, b1_ref, W2_ref, b2_ref,
                    Wlin_ref, blin_ref, o_ref, acc_ref, cnt_ref):
    i = pl.program_id(0)
    g, r = acc_ref.shape[0], h_ref.shape[0]

    @pl.when(i == 0)
    def _():
        acc_ref[...] = jnp.zeros_like(acc_ref)
        cnt_ref[...] = jnp.zeros_like(cnt_ref)

    z = h_ref[...] + agg_ref[0] + agg_ref[1]
    t = jnp.dot(z, W1_ref[...], preferred_element_type=jnp.float32) + b1_ref[...]
    t = jnp.maximum(t, 0.0)
    h2 = jnp.dot(t, W2_ref[...], preferred_element_type=jnp.float32) + b2_ref[...]

    b = batch_ref[0, 0, :]
    gids = lax.broadcasted_iota(jnp.int32, (g, r), 0)
    m = (gids == b[None, :]).astype(jnp.float32)
    acc_ref[...] += jnp.dot(m, h2, preferred_element_type=jnp.float32)
    cnt_ref[...] += jnp.sum(m, axis=1, keepdims=True)

    @pl.when(i == pl.num_programs(0) - 1)
    def _():
        pooled = acc_ref[...] / jnp.maximum(cnt_ref[...], 1.0)
        o_ref[...] = (jnp.dot(pooled, Wlin_ref[...], preferred_element_type=jnp.float32)
                      + blin_ref[...])


def kernel(x, edge_index, batch, W1a, b1a, W2a, b2a, W1b, b1b, W2b, b2b, Wlin, blin):
    n, d = x.shape
    e = edge_index.shape[1]
    g = 128
    o_dim = Wlin.shape[1]
    r = 1000
    n_blocks = n // r

    src = edge_index[0]
    dst = edge_index[1]
    unit = _CHUNK * _NS
    n_pad = ((n + unit - 1) // unit) * unit
    sc_scatter = _make_sc_scatter_add(n_pad, e, d)

    b1a_r, b2a_r = b1a.reshape(1, d), b2a.reshape(1, d)
    b1b_r, b2b_r = b1b.reshape(1, d), b2b.reshape(1, d)
    blin_r = blin.reshape(1, o_dim)
    batch_r = batch.reshape(n_blocks, 1, r)

    agg1 = sc_scatter(x, src, dst)

    h1 = pl.pallas_call(
        _mlp1_body,
        grid=(n_blocks,),
        in_specs=[
            pl.BlockSpec((r, d), lambda i: (i, 0)),
            pl.BlockSpec((_NC, r, d), lambda i: (0, i, 0)),
            pl.BlockSpec((d, d), lambda i: (0, 0)),
            pl.BlockSpec((1, d), lambda i: (0, 0)),
            pl.BlockSpec((d, d), lambda i: (0, 0)),
            pl.BlockSpec((1, d), lambda i: (0, 0)),
        ],
        out_specs=pl.BlockSpec((r, d), lambda i: (i, 0)),
        out_shape=jax.ShapeDtypeStruct((n, d), jnp.float32),
    )(x, agg1, W1a, b1a_r, W2a, b2a_r)

    agg2 = sc_scatter(h1, src, dst)

    out = pl.pallas_call(
        _mlp2_pool_body,
        grid=(n_blocks,),
        in_specs=[
            pl.BlockSpec((1, 1, r), lambda i: (i, 0, 0)),
            pl.BlockSpec((r, d), lambda i: (i, 0)),
            pl.BlockSpec((_NC, r, d), lambda i: (0, i, 0)),
            pl.BlockSpec((d, d), lambda i: (0, 0)),
            pl.BlockSpec((1, d), lambda i: (0, 0)),
            pl.BlockSpec((d, d), lambda i: (0, 0)),
            pl.BlockSpec((1, d), lambda i: (0, 0)),
            pl.BlockSpec((d, o_dim), lambda i: (0, 0)),
            pl.BlockSpec((1, o_dim), lambda i: (0, 0)),
        ],
        out_specs=pl.BlockSpec((g, o_dim), lambda i: (0, 0)),
        out_shape=jax.ShapeDtypeStruct((g, o_dim), jnp.float32),
        scratch_shapes=[
            pltpu.VMEM((g, d), jnp.float32),
            pltpu.VMEM((g, 1), jnp.float32),
        ],
    )(batch_r, h1, agg2, W1b, b1b_r, W2b, b2b_r, Wlin, blin_r)

    return out


# speedup vs baseline: 1.1310x; 1.1310x over previous
import functools

import jax
import jax.numpy as jnp
from jax import lax
from jax.experimental import pallas as pl
from jax.experimental.pallas import tpu as pltpu
from jax.experimental.pallas import tpu_sc as plsc

_NC = 2
_NS = 16
_CHUNK = 80


def _make_sc_scatter_add(n_pad, n_edges, dim):
    epw = n_edges // (_NC * _NS)
    n_chunks = epw // _CHUNK
    rows_per_sub = n_pad // _NS
    zrows = _CHUNK
    n_zero_iters = rows_per_sub // zrows
    mesh = plsc.VectorSubcoreMesh(core_axis_name="c", subcore_axis_name="s")

    @functools.partial(
        pl.kernel,
        out_type=jax.ShapeDtypeStruct((_NC, n_pad, dim), jnp.float32),
        mesh=mesh,
        scratch_types=[
            pltpu.VMEM((epw,), jnp.int32),
            pltpu.VMEM((_CHUNK,), jnp.int32),
            pltpu.VMEM((_CHUNK,), jnp.int32),
            pltpu.VMEM((_CHUNK,), jnp.int32),
            pltpu.VMEM((_CHUNK, dim), jnp.float32),
            pltpu.VMEM((_CHUNK, dim), jnp.float32),
            pltpu.VMEM((_CHUNK, dim), jnp.float32),
            pltpu.VMEM_SHARED((n_pad, dim), jnp.float32),
            pltpu.SemaphoreType.DMA,
            pltpu.SemaphoreType.DMA,
            pltpu.SemaphoreType.DMA,
            pltpu.SemaphoreType.DMA,
            pltpu.SemaphoreType.DMA,
            pltpu.SemaphoreType.DMA,
            pltpu.SemaphoreType.DMA,
            pltpu.SemaphoreType.DMA,
            pltpu.SemaphoreType.DMA,
            pltpu.SemaphoreType.DMA,
        ],
    )
    def k(h_hbm, src_hbm, dst_hbm, out_hbm, src_v, dst_a, dst_b, dst_c,
          rows_a, rows_b, rows_c, acc_sh,
          sem_i, sem_ga, sem_gb, sem_gc, sem_da, sem_db, sem_dc,
          sem_sa, sem_sb, sem_sc):
        c = lax.axis_index("c")
        s = lax.axis_index("s")
        wid = c * _NS + s
        base0 = wid * epw

        dst_bufs = (dst_a, dst_b, dst_c)
        rows_bufs = (rows_a, rows_b, rows_c)
        gsems = (sem_ga, sem_gb, sem_gc)
        dsems = (sem_da, sem_db, sem_dc)
        ssems = (sem_sa, sem_sb, sem_sc)

        def src_idx(t):
            return src_v.at[pl.ds(pl.multiple_of(t * _CHUNK, 8), _CHUNK)]

        def dst_desc(t, j):
            return pltpu.make_async_copy(
                dst_hbm.at[pl.ds(pl.multiple_of(base0 + t * _CHUNK, 8), _CHUNK)],
                dst_bufs[j], dsems[j])

        def gather_desc(t, j):
            return pltpu.make_async_copy(h_hbm.at[src_idx(t)], rows_bufs[j], gsems[j])

        cp_src = pltpu.async_copy(src_hbm.at[pl.ds(wid * epw, epw)], src_v, sem_i)
        dst_desc(0, 0).start()
        dst_desc(1, 1).start()
        dst_desc(2, 2).start()

        zvec = jnp.zeros((1, 16), jnp.float32)

        @pl.loop(0, _CHUNK)
        def _(r):
            @pl.loop(0, dim, step=16)
            def _(j):
                rows_a.at[pl.ds(r, 1), pl.ds(j, 16)][...] = zvec

        @pl.loop(0, n_zero_iters)
        def _(z):
            pltpu.sync_copy(rows_a, acc_sh.at[pl.ds(s * rows_per_sub + z * zrows, zrows)])

        cp_src.wait()
        plsc.subcore_barrier()

        class _Scat:
            def __init__(self, j):
                self.j = j

            def start(self):
                pltpu.async_copy(rows_bufs[self.j], acc_sh.at[dst_bufs[self.j]],
                                 ssems[self.j], add=True)

            def wait(self):
                pltpu.make_async_copy(rows_bufs[self.j], acc_sh.at[dst_bufs[self.j]],
                                      ssems[self.j]).wait()

        def scat_desc(j):
            return _Scat(j)

        gather_desc(0, 0).start()
        gather_desc(1, 1).start()
        gather_desc(2, 2).start()

        def step(t, j, reuse):
            if reuse:
                jp = (j + 2) % 3
                scat_desc(jp).wait()
                dst_desc(t + 2, jp).start()
                gather_desc(t + 2, jp).start()
            gather_desc(t, j).wait()
            dst_desc(t, j).wait()
            scat_desc(j).start()

        step(0, 0, False)

        @pl.loop(0, (n_chunks - 5) // 3)
        def _(kk):
            t = 3 * kk
            step(t + 1, 1, True)
            step(t + 2, 2, True)
            step(t + 3, 0, True)

        step(n_chunks - 4, (n_chunks - 4) % 3, True)
        step(n_chunks - 3, (n_chunks - 3) % 3, True)
        step(n_chunks - 2, (n_chunks - 2) % 3, False)
        step(n_chunks - 1, (n_chunks - 1) % 3, False)

        scat_desc((n_chunks - 3) % 3).wait()
        scat_desc((n_chunks - 2) % 3).wait()
        scat_desc((n_chunks - 1) % 3).wait()

        plsc.subcore_barrier()

        pltpu.sync_copy(
            acc_sh.at[pl.ds(s * rows_per_sub, rows_per_sub)],
            out_hbm.at[c, pl.ds(s * rows_per_sub, rows_per_sub)],
        )

    return k


def _mlp1_body(x_ref, agg_ref, W1_ref, b1_ref, W2_ref, b2_ref, o_ref):
    z = x_ref[...] + agg_ref[0] + agg_ref[1]
    t = jnp.dot(z, W1_ref[...], preferred_element_type=jnp.float32) + b1_ref[...]
    t = jnp.maximum(t, 0.0)
    h = jnp.dot(t, W2_ref[...], preferred_element_type=jnp.float32) + b2_ref[...]
    o_ref[...] = jnp.maximum(h, 0.0)


def _mlp2_pool_body(batch_ref, h_ref, agg_ref, W1_ref, b1_ref, W2_ref, b2_ref,
                    Wlin_ref, blin_ref, o_ref, acc_ref, cnt_ref):
    i = pl.program_id(0)
    g, r = acc_ref.shape[0], h_ref.shape[0]

    @pl.when(i == 0)
    def _():
        acc_ref[...] = jnp.zeros_like(acc_ref)
        cnt_ref[...] = jnp.zeros_like(cnt_ref)

    z = h_ref[...] + agg_ref[0] + agg_ref[1]
    t = jnp.dot(z, W1_ref[...], preferred_element_type=jnp.float32) + b1_ref[...]
    t = jnp.maximum(t, 0.0)
    h2 = jnp.dot(t, W2_ref[...], preferred_element_type=jnp.float32) + b2_ref[...]

    b = batch_ref[0, 0, :]
    gids = lax.broadcasted_iota(jnp.int32, (g, r), 0)
    m = (gids == b[None, :]).astype(jnp.float32)
    acc_ref[...] += jnp.dot(m, h2, preferred_element_type=jnp.float32)
    cnt_ref[...] += jnp.sum(m, axis=1, keepdims=True)

    @pl.when(i == pl.num_programs(0) - 1)
    def _():
        pooled = acc_ref[...] / jnp.maximum(cnt_ref[...], 1.0)
        o_ref[...] = (jnp.dot(pooled, Wlin_ref[...], preferred_element_type=jnp.float32)
                      + blin_ref[...])


def kernel(x, edge_index, batch, W1a, b1a, W2a, b2a, W1b, b1b, W2b, b2b, Wlin, blin):
    n, d = x.shape
    e = edge_index.shape[1]
    g = 128
    o_dim = Wlin.shape[1]
    r = 1000
    n_blocks = n // r

    src = edge_index[0]
    dst = edge_index[1]
    unit = _CHUNK * _NS
    n_pad = ((n + unit - 1) // unit) * unit
    sc_scatter = _make_sc_scatter_add(n_pad, e, d)

    b1a_r, b2a_r = b1a.reshape(1, d), b2a.reshape(1, d)
    b1b_r, b2b_r = b1b.reshape(1, d), b2b.reshape(1, d)
    blin_r = blin.reshape(1, o_dim)
    batch_r = batch.reshape(n_blocks, 1, r)

    agg1 = sc_scatter(x, src, dst)

    h1 = pl.pallas_call(
        _mlp1_body,
        grid=(n_blocks,),
        in_specs=[
            pl.BlockSpec((r, d), lambda i: (i, 0)),
            pl.BlockSpec((_NC, r, d), lambda i: (0, i, 0)),
            pl.BlockSpec((d, d), lambda i: (0, 0)),
            pl.BlockSpec((1, d), lambda i: (0, 0)),
            pl.BlockSpec((d, d), lambda i: (0, 0)),
            pl.BlockSpec((1, d), lambda i: (0, 0)),
        ],
        out_specs=pl.BlockSpec((r, d), lambda i: (i, 0)),
        out_shape=jax.ShapeDtypeStruct((n, d), jnp.float32),
    )(x, agg1, W1a, b1a_r, W2a, b2a_r)

    agg2 = sc_scatter(h1, src, dst)

    out = pl.pallas_call(
        _mlp2_pool_body,
        grid=(n_blocks,),
        in_specs=[
            pl.BlockSpec((1, 1, r), lambda i: (i, 0, 0)),
            pl.BlockSpec((r, d), lambda i: (i, 0)),
            pl.BlockSpec((_NC, r, d), lambda i: (0, i, 0)),
            pl.BlockSpec((d, d), lambda i: (0, 0)),
            pl.BlockSpec((1, d), lambda i: (0, 0)),
            pl.BlockSpec((d, d), lambda i: (0, 0)),
            pl.BlockSpec((1, d), lambda i: (0, 0)),
            pl.BlockSpec((d, o_dim), lambda i: (0, 0)),
            pl.BlockSpec((1, o_dim), lambda i: (0, 0)),
        ],
        out_specs=pl.BlockSpec((g, o_dim), lambda i: (0, 0)),
        out_shape=jax.ShapeDtypeStruct((g, o_dim), jnp.float32),
        scratch_shapes=[
            pltpu.VMEM((g, d), jnp.float32),
            pltpu.VMEM((g, 1), jnp.float32),
        ],
    )(batch_r, h1, agg2, W1b, b1b_r, W2b, b2b_r, Wlin, blin_r)

    return out


# speedup vs baseline: 1.1791x; 1.0426x over previous
import functools

import jax
import jax.numpy as jnp
from jax import lax
from jax.experimental import pallas as pl
from jax.experimental.pallas import tpu as pltpu
from jax.experimental.pallas import tpu_sc as plsc

_NC = 2
_NS = 16
_CHUNK = 80


def _make_sc_scatter_add(n_pad, n_edges, dim):
    epw = n_edges // (_NC * _NS)
    n_chunks = epw // _CHUNK
    rows_per_sub = n_pad // _NS
    zrows = _CHUNK
    n_zero_iters = rows_per_sub // zrows
    mesh = plsc.VectorSubcoreMesh(core_axis_name="c", subcore_axis_name="s")

    @functools.partial(
        pl.kernel,
        out_type=jax.ShapeDtypeStruct((_NC, n_pad, dim), jnp.float32),
        mesh=mesh,
        scratch_types=[
            pltpu.VMEM((epw,), jnp.int32),
            pltpu.VMEM((_CHUNK,), jnp.int32),
            pltpu.VMEM((_CHUNK,), jnp.int32),
            pltpu.VMEM((_CHUNK,), jnp.int32),
            pltpu.VMEM((_CHUNK, dim), jnp.float32),
            pltpu.VMEM((_CHUNK, dim), jnp.float32),
            pltpu.VMEM((_CHUNK, dim), jnp.float32),
            pltpu.VMEM_SHARED((n_pad, dim), jnp.float32),
            pltpu.SemaphoreType.DMA,
            pltpu.SemaphoreType.DMA,
            pltpu.SemaphoreType.DMA,
            pltpu.SemaphoreType.DMA,
            pltpu.SemaphoreType.DMA,
            pltpu.SemaphoreType.DMA,
            pltpu.SemaphoreType.DMA,
            pltpu.SemaphoreType.DMA,
            pltpu.SemaphoreType.DMA,
            pltpu.SemaphoreType.DMA,
        ],
    )
    def k(h_hbm, ei_hbm, out_hbm, src_v, dst_a, dst_b, dst_c,
          rows_a, rows_b, rows_c, acc_sh,
          sem_i, sem_ga, sem_gb, sem_gc, sem_da, sem_db, sem_dc,
          sem_sa, sem_sb, sem_sc):
        c = lax.axis_index("c")
        s = lax.axis_index("s")
        wid = c * _NS + s
        base0 = wid * epw

        dst_bufs = (dst_a, dst_b, dst_c)
        rows_bufs = (rows_a, rows_b, rows_c)
        gsems = (sem_ga, sem_gb, sem_gc)
        dsems = (sem_da, sem_db, sem_dc)
        ssems = (sem_sa, sem_sb, sem_sc)

        def src_idx(t):
            return src_v.at[pl.ds(pl.multiple_of(t * _CHUNK, 8), _CHUNK)]

        def dst_desc(t, j):
            return pltpu.make_async_copy(
                ei_hbm.at[pl.ds(pl.multiple_of(n_edges + base0 + t * _CHUNK, 8), _CHUNK)],
                dst_bufs[j], dsems[j])

        def gather_desc(t, j):
            return pltpu.make_async_copy(h_hbm.at[src_idx(t)], rows_bufs[j], gsems[j])

        cp_src = pltpu.async_copy(ei_hbm.at[pl.ds(wid * epw, epw)], src_v, sem_i)
        dst_desc(0, 0).start()
        dst_desc(1, 1).start()
        dst_desc(2, 2).start()

        zvec = jnp.zeros((1, 16), jnp.float32)

        @pl.loop(0, _CHUNK)
        def _(r):
            @pl.loop(0, dim, step=16)
            def _(j):
                rows_a.at[pl.ds(r, 1), pl.ds(j, 16)][...] = zvec

        @pl.loop(0, n_zero_iters)
        def _(z):
            pltpu.sync_copy(rows_a, acc_sh.at[pl.ds(s * rows_per_sub + z * zrows, zrows)])

        cp_src.wait()
        plsc.subcore_barrier()

        class _Scat:
            def __init__(self, j):
                self.j = j

            def start(self):
                pltpu.async_copy(rows_bufs[self.j], acc_sh.at[dst_bufs[self.j]],
                                 ssems[self.j], add=True)

            def wait(self):
                pltpu.make_async_copy(rows_bufs[self.j], acc_sh.at[dst_bufs[self.j]],
                                      ssems[self.j]).wait()

        def scat_desc(j):
            return _Scat(j)

        gather_desc(0, 0).start()
        gather_desc(1, 1).start()
        gather_desc(2, 2).start()

        def step(t, j, reuse):
            if reuse:
                jp = (j + 2) % 3
                scat_desc(jp).wait()
                dst_desc(t + 2, jp).start()
                gather_desc(t + 2, jp).start()
            gather_desc(t, j).wait()
            dst_desc(t, j).wait()
            scat_desc(j).start()

        step(0, 0, False)

        @pl.loop(0, (n_chunks - 5) // 3)
        def _(kk):
            t = 3 * kk
            step(t + 1, 1, True)
            step(t + 2, 2, True)
            step(t + 3, 0, True)

        step(n_chunks - 4, (n_chunks - 4) % 3, True)
        step(n_chunks - 3, (n_chunks - 3) % 3, True)
        step(n_chunks - 2, (n_chunks - 2) % 3, False)
        step(n_chunks - 1, (n_chunks - 1) % 3, False)

        scat_desc((n_chunks - 3) % 3).wait()
        scat_desc((n_chunks - 2) % 3).wait()
        scat_desc((n_chunks - 1) % 3).wait()

        plsc.subcore_barrier()

        pltpu.sync_copy(
            acc_sh.at[pl.ds(s * rows_per_sub, rows_per_sub)],
            out_hbm.at[c, pl.ds(s * rows_per_sub, rows_per_sub)],
        )

    return k


def _mlp1_body(x_ref, agg_ref, W1_ref, b1_ref, W2_ref, b2_ref, o_ref):
    z = x_ref[...] + agg_ref[0] + agg_ref[1]
    t = jnp.dot(z, W1_ref[...], preferred_element_type=jnp.float32) + b1_ref[...]
    t = jnp.maximum(t, 0.0)
    h = jnp.dot(t, W2_ref[...], preferred_element_type=jnp.float32) + b2_ref[...]
    o_ref[...] = jnp.maximum(h, 0.0)


def _mlp2_pool_body(batch_ref, h_ref, agg_ref, W1_ref, b1_ref, W2_ref, b2_ref,
                    Wlin_ref, blin_ref, o_ref, acc_ref, cnt_ref):
    i = pl.program_id(0)
    g, r = acc_ref.shape[0], h_ref.shape[0]

    @pl.when(i == 0)
    def _():
        acc_ref[...] = jnp.zeros_like(acc_ref)
        cnt_ref[...] = jnp.zeros_like(cnt_ref)

    z = h_ref[...] + agg_ref[0] + agg_ref[1]
    t = jnp.dot(z, W1_ref[...], preferred_element_type=jnp.float32) + b1_ref[...]
    t = jnp.maximum(t, 0.0)
    h2 = jnp.dot(t, W2_ref[...], preferred_element_type=jnp.float32) + b2_ref[...]

    b = batch_ref[0, 0, :]
    gids = lax.broadcasted_iota(jnp.int32, (g, r), 0)
    m = (gids == b[None, :]).astype(jnp.float32)
    acc_ref[...] += jnp.dot(m, h2, preferred_element_type=jnp.float32)
    cnt_ref[...] += jnp.sum(m, axis=1, keepdims=True)

    @pl.when(i == pl.num_programs(0) - 1)
    def _():
        pooled = acc_ref[...] / jnp.maximum(cnt_ref[...], 1.0)
        o_ref[...] = (jnp.dot(pooled, Wlin_ref[...], preferred_element_type=jnp.float32)
                      + blin_ref[...])


def kernel(x, edge_index, batch, W1a, b1a, W2a, b2a, W1b, b1b, W2b, b2b, Wlin, blin):
    n, d = x.shape
    e = edge_index.shape[1]
    g = 128
    o_dim = Wlin.shape[1]
    r = 1000
    n_blocks = n // r

    ei_flat = edge_index.reshape(-1)
    unit = _CHUNK * _NS
    n_pad = ((n + unit - 1) // unit) * unit
    sc_scatter = _make_sc_scatter_add(n_pad, e, d)

    b1a_r, b2a_r = b1a.reshape(1, d), b2a.reshape(1, d)
    b1b_r, b2b_r = b1b.reshape(1, d), b2b.reshape(1, d)
    blin_r = blin.reshape(1, o_dim)
    batch_r = batch.reshape(n_blocks, 1, r)

    agg1 = sc_scatter(x, ei_flat)

    h1 = pl.pallas_call(
        _mlp1_body,
        grid=(n_blocks,),
        in_specs=[
            pl.BlockSpec((r, d), lambda i: (i, 0)),
            pl.BlockSpec((_NC, r, d), lambda i: (0, i, 0)),
            pl.BlockSpec((d, d), lambda i: (0, 0)),
            pl.BlockSpec((1, d), lambda i: (0, 0)),
            pl.BlockSpec((d, d), lambda i: (0, 0)),
            pl.BlockSpec((1, d), lambda i: (0, 0)),
        ],
        out_specs=pl.BlockSpec((r, d), lambda i: (i, 0)),
        out_shape=jax.ShapeDtypeStruct((n, d), jnp.float32),
    )(x, agg1, W1a, b1a_r, W2a, b2a_r)

    agg2 = sc_scatter(h1, ei_flat)

    out = pl.pallas_call(
        _mlp2_pool_body,
        grid=(n_blocks,),
        in_specs=[
            pl.BlockSpec((1, 1, r), lambda i: (i, 0, 0)),
            pl.BlockSpec((r, d), lambda i: (i, 0)),
            pl.BlockSpec((_NC, r, d), lambda i: (0, i, 0)),
            pl.BlockSpec((d, d), lambda i: (0, 0)),
            pl.BlockSpec((1, d), lambda i: (0, 0)),
            pl.BlockSpec((d, d), lambda i: (0, 0)),
            pl.BlockSpec((1, d), lambda i: (0, 0)),
            pl.BlockSpec((d, o_dim), lambda i: (0, 0)),
            pl.BlockSpec((1, o_dim), lambda i: (0, 0)),
        ],
        out_specs=pl.BlockSpec((g, o_dim), lambda i: (0, 0)),
        out_shape=jax.ShapeDtypeStruct((g, o_dim), jnp.float32),
        scratch_shapes=[
            pltpu.VMEM((g, d), jnp.float32),
            pltpu.VMEM((g, 1), jnp.float32),
        ],
    )(batch_r, h1, agg2, W1b, b1b_r, W2b, b2b_r, Wlin, blin_r)

    return out
